# baseline (speedup 1.0000x reference)
import functools

import jax
import jax.numpy as jnp
from jax import lax
from jax.experimental import pallas as pl
from jax.experimental.pallas import tpu as pltpu
from jax.experimental.pallas import tpu_sc as plsc

NC = 2
NS = 16
NW = NC * NS
K = 128

F32 = jnp.float32


def _sc_mesh():
    return plsc.VectorSubcoreMesh(
        core_axis_name="c", subcore_axis_name="s", num_cores=NC, num_subcores=NS
    )


def _sc_hist(dst2d, ones_blk, zeros_h):
    C = dst2d.shape[0]
    NPAD = zeros_h.shape[0]
    rps = NPAD // NS
    n_iter = pl.cdiv(C, NW)

    @functools.partial(
        pl.kernel,
        out_type=jax.ShapeDtypeStruct((NC * NPAD, 128), F32),
        mesh=_sc_mesh(),
        scratch_types=[
            pltpu.VMEM_SHARED((NPAD, 128), F32),
            pltpu.VMEM((K,), jnp.int32),
            pltpu.VMEM((K, 128), F32),
        ],
    )
    def k(dst_h, ones_h, zeros_hbm, out_h, acc, idx_v, ones_v):
        cid = lax.axis_index("c")
        sid = lax.axis_index("s")
        wid = sid * NC + cid
        base = sid * rps
        pltpu.sync_copy(zeros_hbm.at[pl.ds(base, rps)], acc.at[pl.ds(base, rps)])
        pltpu.sync_copy(ones_h, ones_v)
        plsc.subcore_barrier()

        @pl.loop(0, n_iter)
        def _(j):
            c = wid + j * NW

            @pl.when(c < C)
            def _():
                pltpu.sync_copy(dst_h.at[c], idx_v)
                pltpu.sync_copy(ones_v, acc.at[idx_v], add=True)

        plsc.subcore_barrier()
        pltpu.sync_copy(
            acc.at[pl.ds(base, rps)], out_h.at[pl.ds(cid * NPAD + base, rps)]
        )

    return k(dst2d, ones_blk, zeros_h)


def _sc_scatter(v, src2d, dst2d, zeros_h):
    C = src2d.shape[0]
    NPAD, D = zeros_h.shape
    rps = NPAD // NS
    n_iter = pl.cdiv(C, NW)

    @functools.partial(
        pl.kernel,
        out_type=jax.ShapeDtypeStruct((NC * NPAD, D), F32),
        mesh=_sc_mesh(),
        scratch_types=[
            pltpu.VMEM_SHARED((NPAD, D), F32),
            pltpu.VMEM((K,), jnp.int32),
            pltpu.VMEM((K,), jnp.int32),
            pltpu.VMEM((K, D), F32),
        ],
    )
    def k(v_h, src_h, dst_h, zeros_hbm, out_h, acc, isrc, idst, rows):
        cid = lax.axis_index("c")
        sid = lax.axis_index("s")
        wid = sid * NC + cid
        base = sid * rps
        pltpu.sync_copy(zeros_hbm.at[pl.ds(base, rps)], acc.at[pl.ds(base, rps)])
        plsc.subcore_barrier()

        @pl.loop(0, n_iter)
        def _(j):
            c = wid + j * NW

            @pl.when(c < C)
            def _():
                pltpu.sync_copy(src_h.at[c], isrc)
                pltpu.sync_copy(dst_h.at[c], idst)
                pltpu.sync_copy(v_h.at[isrc], rows)
                pltpu.sync_copy(rows, acc.at[idst], add=True)

        plsc.subcore_barrier()
        pltpu.sync_copy(
            acc.at[pl.ds(base, rps)], out_h.at[pl.ds(cid * NPAD + base, rps)]
        )

    return k(v, src2d, dst2d, zeros_h)


def _tc1(histp, x, rb):
    n = x.shape[0]
    grid = (n // rb,)

    def body(h_ref, x_ref, v1_ref, dinv_ref):
        deg = h_ref[0, :, 0:1] + h_ref[1, :, 0:1] + 1.0
        dinv = lax.rsqrt(deg)
        dinv_ref[...] = dinv
        v1_ref[...] = x_ref[...] * dinv

    return pl.pallas_call(
        body,
        grid=grid,
        in_specs=[
            pl.BlockSpec((NC, rb, 128), lambda i: (0, i, 0)),
            pl.BlockSpec((rb, 128), lambda i: (i, 0)),
        ],
        out_specs=[
            pl.BlockSpec((rb, 128), lambda i: (i, 0)),
            pl.BlockSpec((rb, 1), lambda i: (i, 0)),
        ],
        out_shape=[
            jax.ShapeDtypeStruct((n, 128), F32),
            jax.ShapeDtypeStruct((n, 1), F32),
        ],
    )(histp, x)


def _tc2(y1p, v1, dinv, W1, b1, W2, rb):
    n = v1.shape[0]
    grid = (n // rb,)

    def body(y_ref, v1_ref, dinv_ref, w1_ref, b1_ref, w2_ref, v2_ref):
        dinv = dinv_ref[...]
        t = (y_ref[0] + y_ref[1] + v1_ref[...]) * dinv
        h = jnp.dot(
            t, w1_ref[...], preferred_element_type=F32,
            precision=lax.Precision.HIGHEST,
        ) + b1_ref[...]
        h = jnp.maximum(h, 0.0)
        g = jnp.dot(
            h, w2_ref[...], preferred_element_type=F32,
            precision=lax.Precision.HIGHEST,
        )
        v2_ref[...] = g * dinv

    return pl.pallas_call(
        body,
        grid=grid,
        in_specs=[
            pl.BlockSpec((NC, rb, 128), lambda i: (0, i, 0)),
            pl.BlockSpec((rb, 128), lambda i: (i, 0)),
            pl.BlockSpec((rb, 1), lambda i: (i, 0)),
            pl.BlockSpec((128, 256), lambda i: (0, 0)),
            pl.BlockSpec((1, 256), lambda i: (0, 0)),
            pl.BlockSpec((256, 128), lambda i: (0, 0)),
        ],
        out_specs=pl.BlockSpec((rb, 128), lambda i: (i, 0)),
        out_shape=jax.ShapeDtypeStruct((n, 128), F32),
    )(y1p, v1, dinv, W1, b1, W2)


def _tc3(y2p, v2, dinv, b2, rb):
    n = v2.shape[0]
    grid = (n // rb,)

    def body(y_ref, v2_ref, dinv_ref, b2_ref, o_ref):
        o_ref[...] = (
            (y_ref[0] + y_ref[1] + v2_ref[...]) * dinv_ref[...] + b2_ref[...]
        )

    return pl.pallas_call(
        body,
        grid=grid,
        in_specs=[
            pl.BlockSpec((NC, rb, 128), lambda i: (0, i, 0)),
            pl.BlockSpec((rb, 128), lambda i: (i, 0)),
            pl.BlockSpec((rb, 1), lambda i: (i, 0)),
            pl.BlockSpec((1, 128), lambda i: (0, 0)),
        ],
        out_specs=pl.BlockSpec((rb, 128), lambda i: (i, 0)),
        out_shape=jax.ShapeDtypeStruct((n, 128), F32),
    )(y2p, v2, dinv, b2)


def kernel(x, edge_index, W1, b1, W2, b2):
    n = x.shape[0]
    e = edge_index.shape[1]
    assert e % K == 0
    c = e // K
    npad = ((n + NW * 8 - 1) // (NW * 8)) * (NW * 8)
    rb = 2000

    ei = edge_index.astype(jnp.int32)
    src2d = ei[0].reshape(c, K)
    dst2d = ei[1].reshape(c, K)
    zeros128 = jnp.zeros((npad, 128), F32)
    ones_blk = jnp.ones((K, 128), F32)
    b1r = b1.reshape(1, -1)
    b2r = b2.reshape(1, -1)

    histp = _sc_hist(dst2d, ones_blk, zeros128).reshape(NC, npad, 128)
    v1, dinv = _tc1(histp, x, rb)
    y1p = _sc_scatter(v1, src2d, dst2d, zeros128).reshape(NC, npad, 128)
    v2 = _tc2(y1p, v1, dinv, W1, b1r, W2, rb)
    y2p = _sc_scatter(v2, src2d, dst2d, zeros128).reshape(NC, npad, 128)
    out = _tc3(y2p, v2, dinv, b2r, rb)
    return out


# speedup vs baseline: 1.0664x; 1.0664x over previous
import functools

import jax
import jax.numpy as jnp
from jax import lax
from jax.experimental import pallas as pl
from jax.experimental.pallas import tpu as pltpu
from jax.experimental.pallas import tpu_sc as plsc

NC = 2
NS = 16
NW = NC * NS
K = 128

F32 = jnp.float32


def _sc_mesh():
    return plsc.VectorSubcoreMesh(
        core_axis_name="c", subcore_axis_name="s", num_cores=NC, num_subcores=NS
    )


def _sc_hist(dst2d, ones_blk, zeros_h):
    C = dst2d.shape[0]
    NPAD = zeros_h.shape[0]
    rps = NPAD // NS
    n_iter = pl.cdiv(C, NW)

    @functools.partial(
        pl.kernel,
        out_type=jax.ShapeDtypeStruct((NC * NPAD, 128), F32),
        mesh=_sc_mesh(),
        scratch_types=[
            pltpu.VMEM_SHARED((NPAD, 128), F32),
            pltpu.VMEM((K,), jnp.int32),
            pltpu.VMEM((K, 128), F32),
        ],
    )
    def k(dst_h, ones_h, zeros_hbm, out_h, acc, idx_v, ones_v):
        cid = lax.axis_index("c")
        sid = lax.axis_index("s")
        wid = sid * NC + cid
        base = sid * rps
        pltpu.sync_copy(zeros_hbm.at[pl.ds(base, rps)], acc.at[pl.ds(base, rps)])
        pltpu.sync_copy(ones_h, ones_v)
        plsc.subcore_barrier()

        @pl.loop(0, n_iter)
        def _(j):
            c = wid + j * NW

            @pl.when(c < C)
            def _():
                pltpu.sync_copy(dst_h.at[c], idx_v)
                pltpu.sync_copy(ones_v, acc.at[idx_v], add=True)

        plsc.subcore_barrier()
        pltpu.sync_copy(
            acc.at[pl.ds(base, rps)], out_h.at[pl.ds(cid * NPAD + base, rps)]
        )

    return k(dst2d, ones_blk, zeros_h)


def _sc_scatter(v, src2d, dst2d, zeros_h):
    C = src2d.shape[0]
    NPAD, D = zeros_h.shape
    rps = NPAD // NS
    n_iter = pl.cdiv(C, NW)

    n_pairs = pl.cdiv(C, 2 * NW)

    @functools.partial(
        pl.kernel,
        out_type=jax.ShapeDtypeStruct((NC * NPAD, D), F32),
        mesh=_sc_mesh(),
        scratch_types=[
            pltpu.VMEM_SHARED((NPAD, D), F32),
            pltpu.VMEM((K,), jnp.int32),
            pltpu.VMEM((K,), jnp.int32),
            pltpu.VMEM((K,), jnp.int32),
            pltpu.VMEM((K,), jnp.int32),
            pltpu.VMEM((K, D), F32),
            pltpu.VMEM((K, D), F32),
            pltpu.SemaphoreType.DMA,
            pltpu.SemaphoreType.DMA,
            pltpu.SemaphoreType.DMA,
            pltpu.SemaphoreType.DMA,
        ],
    )
    def k(v_h, src_h, dst_h, zeros_hbm, out_h, acc, isrc0, idst0, isrc1, idst1,
          r0, r1, s0, s1, s2, s3):
        cid = lax.axis_index("c")
        sid = lax.axis_index("s")
        wid = sid * NC + cid
        base = sid * rps
        pltpu.sync_copy(zeros_hbm.at[pl.ds(base, rps)], acc.at[pl.ds(base, rps)])
        plsc.subcore_barrier()

        @pl.loop(0, n_pairs)
        def _(p):
            c0 = wid + (2 * p) * NW
            c1 = c0 + NW

            @pl.when(c1 < C)
            def _():
                pltpu.sync_copy(src_h.at[c0], isrc0)
                pltpu.sync_copy(dst_h.at[c0], idst0)
                pltpu.sync_copy(src_h.at[c1], isrc1)
                pltpu.sync_copy(dst_h.at[c1], idst1)
                d0 = pltpu.async_copy(v_h.at[isrc0], r0, s0)
                d1 = pltpu.async_copy(v_h.at[isrc1], r1, s1)
                d0.wait()
                e0 = pltpu.async_copy(r0, acc.at[idst0], s2, add=True)
                d1.wait()
                e1 = pltpu.async_copy(r1, acc.at[idst1], s3, add=True)
                e0.wait()
                e1.wait()

            @pl.when((c0 < C) & (c1 >= C))
            def _():
                pltpu.sync_copy(src_h.at[c0], isrc0)
                pltpu.sync_copy(dst_h.at[c0], idst0)
                pltpu.sync_copy(v_h.at[isrc0], r0)
                pltpu.sync_copy(r0, acc.at[idst0], add=True)

        plsc.subcore_barrier()
        pltpu.sync_copy(
            acc.at[pl.ds(base, rps)], out_h.at[pl.ds(cid * NPAD + base, rps)]
        )

    return k(v, src2d, dst2d, zeros_h)


def _tc1(histp, x, rb):
    n = x.shape[0]
    grid = (n // rb,)

    def body(h_ref, x_ref, v1_ref, dinv_ref):
        deg = h_ref[0, :, 0:1] + h_ref[1, :, 0:1] + 1.0
        dinv = lax.rsqrt(deg)
        dinv_ref[...] = dinv
        v1_ref[...] = x_ref[...] * dinv

    return pl.pallas_call(
        body,
        grid=grid,
        in_specs=[
            pl.BlockSpec((NC, rb, 128), lambda i: (0, i, 0)),
            pl.BlockSpec((rb, 128), lambda i: (i, 0)),
        ],
        out_specs=[
            pl.BlockSpec((rb, 128), lambda i: (i, 0)),
            pl.BlockSpec((rb, 1), lambda i: (i, 0)),
        ],
        out_shape=[
            jax.ShapeDtypeStruct((n, 128), F32),
            jax.ShapeDtypeStruct((n, 1), F32),
        ],
    )(histp, x)


def _tc2(y1p, v1, dinv, W1, b1, W2, rb):
    n = v1.shape[0]
    grid = (n // rb,)

    def body(y_ref, v1_ref, dinv_ref, w1_ref, b1_ref, w2_ref, v2_ref):
        dinv = dinv_ref[...]
        t = (y_ref[0] + y_ref[1] + v1_ref[...]) * dinv
        h = jnp.dot(
            t, w1_ref[...], preferred_element_type=F32,
            precision=lax.Precision.HIGHEST,
        ) + b1_ref[...]
        h = jnp.maximum(h, 0.0)
        g = jnp.dot(
            h, w2_ref[...], preferred_element_type=F32,
            precision=lax.Precision.HIGHEST,
        )
        v2_ref[...] = g * dinv

    return pl.pallas_call(
        body,
        grid=grid,
        in_specs=[
            pl.BlockSpec((NC, rb, 128), lambda i: (0, i, 0)),
            pl.BlockSpec((rb, 128), lambda i: (i, 0)),
            pl.BlockSpec((rb, 1), lambda i: (i, 0)),
            pl.BlockSpec((128, 256), lambda i: (0, 0)),
            pl.BlockSpec((1, 256), lambda i: (0, 0)),
            pl.BlockSpec((256, 128), lambda i: (0, 0)),
        ],
        out_specs=pl.BlockSpec((rb, 128), lambda i: (i, 0)),
        out_shape=jax.ShapeDtypeStruct((n, 128), F32),
    )(y1p, v1, dinv, W1, b1, W2)


def _tc3(y2p, v2, dinv, b2, rb):
    n = v2.shape[0]
    grid = (n // rb,)

    def body(y_ref, v2_ref, dinv_ref, b2_ref, o_ref):
        o_ref[...] = (
            (y_ref[0] + y_ref[1] + v2_ref[...]) * dinv_ref[...] + b2_ref[...]
        )

    return pl.pallas_call(
        body,
        grid=grid,
        in_specs=[
            pl.BlockSpec((NC, rb, 128), lambda i: (0, i, 0)),
            pl.BlockSpec((rb, 128), lambda i: (i, 0)),
            pl.BlockSpec((rb, 1), lambda i: (i, 0)),
            pl.BlockSpec((1, 128), lambda i: (0, 0)),
        ],
        out_specs=pl.BlockSpec((rb, 128), lambda i: (i, 0)),
        out_shape=jax.ShapeDtypeStruct((n, 128), F32),
    )(y2p, v2, dinv, b2)


def kernel(x, edge_index, W1, b1, W2, b2):
    n = x.shape[0]
    e = edge_index.shape[1]
    assert e % K == 0
    c = e // K
    npad = ((n + NW * 8 - 1) // (NW * 8)) * (NW * 8)
    rb = 2000

    ei = edge_index.astype(jnp.int32)
    src2d = ei[0].reshape(c, K)
    dst2d = ei[1].reshape(c, K)
    zeros128 = jnp.zeros((npad, 128), F32)
    ones_blk = jnp.ones((K, 128), F32)
    b1r = b1.reshape(1, -1)
    b2r = b2.reshape(1, -1)

    histp = _sc_hist(dst2d, ones_blk, zeros128).reshape(NC, npad, 128)
    v1, dinv = _tc1(histp, x, rb)
    y1p = _sc_scatter(v1, src2d, dst2d, zeros128).reshape(NC, npad, 128)
    v2 = _tc2(y1p, v1, dinv, W1, b1r, W2, rb)
    y2p = _sc_scatter(v2, src2d, dst2d, zeros128).reshape(NC, npad, 128)
    out = _tc3(y2p, v2, dinv, b2r, rb)
    return out


# speedup vs baseline: 1.3533x; 1.2691x over previous
import functools

import jax
import jax.numpy as jnp
from jax import lax
from jax.experimental import pallas as pl
from jax.experimental.pallas import tpu as pltpu
from jax.experimental.pallas import tpu_sc as plsc

NC = 2
NS = 16
NW = NC * NS
K = 128

F32 = jnp.float32


def _sc_mesh():
    return plsc.VectorSubcoreMesh(
        core_axis_name="c", subcore_axis_name="s", num_cores=NC, num_subcores=NS
    )


def _sc_hist(dst2d, ones_blk, zeros_h):
    C = dst2d.shape[0]
    NPAD = zeros_h.shape[0]
    rps = NPAD // NS
    n_pairs = pl.cdiv(C, 2 * NW)

    @functools.partial(
        pl.kernel,
        out_type=jax.ShapeDtypeStruct((NC * NPAD, 128), F32),
        mesh=_sc_mesh(),
        scratch_types=[
            pltpu.VMEM_SHARED((NPAD, 128), F32),
            pltpu.VMEM((K,), jnp.int32),
            pltpu.VMEM((K,), jnp.int32),
            pltpu.VMEM((K, 128), F32),
            pltpu.SemaphoreType.DMA,
            pltpu.SemaphoreType.DMA,
            pltpu.SemaphoreType.DMA,
            pltpu.SemaphoreType.DMA,
        ],
    )
    def k(dst_h, ones_h, zeros_hbm, out_h, acc, idx0, idx1, ones_v,
          s0, s1, si0, si1):
        cid = lax.axis_index("c")
        sid = lax.axis_index("s")
        wid = sid * NC + cid
        base = sid * rps
        pltpu.sync_copy(zeros_hbm.at[pl.ds(base, rps)], acc.at[pl.ds(base, rps)])
        pltpu.sync_copy(ones_h, ones_v)
        plsc.subcore_barrier()

        @pl.loop(0, n_pairs)
        def _(p):
            c0 = wid + (2 * p) * NW
            c1 = c0 + NW

            @pl.when(c1 < C)
            def _():
                i0 = pltpu.async_copy(dst_h.at[c0], idx0, si0)
                i1 = pltpu.async_copy(dst_h.at[c1], idx1, si1)
                i0.wait()
                e0 = pltpu.async_copy(ones_v, acc.at[idx0], s0, add=True)
                i1.wait()
                e1 = pltpu.async_copy(ones_v, acc.at[idx1], s1, add=True)
                e0.wait()
                e1.wait()

            @pl.when((c0 < C) & (c1 >= C))
            def _():
                pltpu.sync_copy(dst_h.at[c0], idx0)
                pltpu.sync_copy(ones_v, acc.at[idx0], add=True)

        plsc.subcore_barrier()
        pltpu.sync_copy(
            acc.at[pl.ds(base, rps)], out_h.at[pl.ds(cid * NPAD + base, rps)]
        )

    return k(dst2d, ones_blk, zeros_h)


def _sc_scatter(v, src2d, dst2d, zeros_h):
    C = src2d.shape[0]
    NPAD, D = zeros_h.shape
    rps = NPAD // NS
    n_iter = pl.cdiv(C, NW)

    n_pairs = pl.cdiv(C, 2 * NW)

    @functools.partial(
        pl.kernel,
        out_type=jax.ShapeDtypeStruct((NC * NPAD, D), F32),
        mesh=_sc_mesh(),
        scratch_types=[
            pltpu.VMEM_SHARED((NPAD, D), F32),
            pltpu.VMEM((K,), jnp.int32),
            pltpu.VMEM((K,), jnp.int32),
            pltpu.VMEM((K,), jnp.int32),
            pltpu.VMEM((K,), jnp.int32),
            pltpu.VMEM((K, D), F32),
            pltpu.VMEM((K, D), F32),
            pltpu.SemaphoreType.DMA,
            pltpu.SemaphoreType.DMA,
            pltpu.SemaphoreType.DMA,
            pltpu.SemaphoreType.DMA,
            pltpu.SemaphoreType.DMA,
            pltpu.SemaphoreType.DMA,
            pltpu.SemaphoreType.DMA,
            pltpu.SemaphoreType.DMA,
        ],
    )
    def k(v_h, src_h, dst_h, zeros_hbm, out_h, acc, isrc0, idst0, isrc1, idst1,
          r0, r1, s0, s1, s2, s3, si0, si1, si2, si3):
        cid = lax.axis_index("c")
        sid = lax.axis_index("s")
        wid = sid * NC + cid
        base = sid * rps
        pltpu.sync_copy(zeros_hbm.at[pl.ds(base, rps)], acc.at[pl.ds(base, rps)])
        plsc.subcore_barrier()

        @pl.loop(0, n_pairs)
        def _(p):
            c0 = wid + (2 * p) * NW
            c1 = c0 + NW

            @pl.when(c1 < C)
            def _():
                i0 = pltpu.async_copy(src_h.at[c0], isrc0, si0)
                i1 = pltpu.async_copy(dst_h.at[c0], idst0, si1)
                i2 = pltpu.async_copy(src_h.at[c1], isrc1, si2)
                i3 = pltpu.async_copy(dst_h.at[c1], idst1, si3)
                i0.wait()
                d0 = pltpu.async_copy(v_h.at[isrc0], r0, s0)
                i2.wait()
                d1 = pltpu.async_copy(v_h.at[isrc1], r1, s1)
                d0.wait()
                i1.wait()
                e0 = pltpu.async_copy(r0, acc.at[idst0], s2, add=True)
                d1.wait()
                i3.wait()
                e1 = pltpu.async_copy(r1, acc.at[idst1], s3, add=True)
                e0.wait()
                e1.wait()

            @pl.when((c0 < C) & (c1 >= C))
            def _():
                pltpu.sync_copy(src_h.at[c0], isrc0)
                pltpu.sync_copy(dst_h.at[c0], idst0)
                pltpu.sync_copy(v_h.at[isrc0], r0)
                pltpu.sync_copy(r0, acc.at[idst0], add=True)

        plsc.subcore_barrier()
        pltpu.sync_copy(
            acc.at[pl.ds(base, rps)], out_h.at[pl.ds(cid * NPAD + base, rps)]
        )

    return k(v, src2d, dst2d, zeros_h)


def _tc1(histp, x, rb):
    n = x.shape[0]
    grid = (n // rb,)

    def body(h_ref, x_ref, v1_ref, dinv_ref):
        deg = h_ref[0, :, 0:1] + h_ref[1, :, 0:1] + 1.0
        dinv = lax.rsqrt(deg)
        dinv_ref[...] = dinv
        v1_ref[...] = x_ref[...] * dinv

    return pl.pallas_call(
        body,
        grid=grid,
        in_specs=[
            pl.BlockSpec((NC, rb, 128), lambda i: (0, i, 0)),
            pl.BlockSpec((rb, 128), lambda i: (i, 0)),
        ],
        out_specs=[
            pl.BlockSpec((rb, 128), lambda i: (i, 0)),
            pl.BlockSpec((rb, 1), lambda i: (i, 0)),
        ],
        out_shape=[
            jax.ShapeDtypeStruct((n, 128), F32),
            jax.ShapeDtypeStruct((n, 1), F32),
        ],
    )(histp, x)


def _tc2(y1p, v1, dinv, W1, b1, W2, rb):
    n = v1.shape[0]
    grid = (n // rb,)

    def body(y_ref, v1_ref, dinv_ref, w1_ref, b1_ref, w2_ref, v2_ref):
        dinv = dinv_ref[...]
        t = (y_ref[0] + y_ref[1] + v1_ref[...]) * dinv
        h = jnp.dot(
            t, w1_ref[...], preferred_element_type=F32,
            precision=lax.Precision.HIGHEST,
        ) + b1_ref[...]
        h = jnp.maximum(h, 0.0)
        g = jnp.dot(
            h, w2_ref[...], preferred_element_type=F32,
            precision=lax.Precision.HIGHEST,
        )
        v2_ref[...] = g * dinv

    return pl.pallas_call(
        body,
        grid=grid,
        in_specs=[
            pl.BlockSpec((NC, rb, 128), lambda i: (0, i, 0)),
            pl.BlockSpec((rb, 128), lambda i: (i, 0)),
            pl.BlockSpec((rb, 1), lambda i: (i, 0)),
            pl.BlockSpec((128, 256), lambda i: (0, 0)),
            pl.BlockSpec((1, 256), lambda i: (0, 0)),
            pl.BlockSpec((256, 128), lambda i: (0, 0)),
        ],
        out_specs=pl.BlockSpec((rb, 128), lambda i: (i, 0)),
        out_shape=jax.ShapeDtypeStruct((n, 128), F32),
    )(y1p, v1, dinv, W1, b1, W2)


def _tc3(y2p, v2, dinv, b2, rb):
    n = v2.shape[0]
    grid = (n // rb,)

    def body(y_ref, v2_ref, dinv_ref, b2_ref, o_ref):
        o_ref[...] = (
            (y_ref[0] + y_ref[1] + v2_ref[...]) * dinv_ref[...] + b2_ref[...]
        )

    return pl.pallas_call(
        body,
        grid=grid,
        in_specs=[
            pl.BlockSpec((NC, rb, 128), lambda i: (0, i, 0)),
            pl.BlockSpec((rb, 128), lambda i: (i, 0)),
            pl.BlockSpec((rb, 1), lambda i: (i, 0)),
            pl.BlockSpec((1, 128), lambda i: (0, 0)),
        ],
        out_specs=pl.BlockSpec((rb, 128), lambda i: (i, 0)),
        out_shape=jax.ShapeDtypeStruct((n, 128), F32),
    )(y2p, v2, dinv, b2)


def kernel(x, edge_index, W1, b1, W2, b2):
    n = x.shape[0]
    e = edge_index.shape[1]
    assert e % K == 0
    c = e // K
    npad = ((n + NW * 8 - 1) // (NW * 8)) * (NW * 8)
    rb = 2000

    ei = edge_index.astype(jnp.int32)
    src2d = ei[0].reshape(c, K)
    dst2d = ei[1].reshape(c, K)
    zeros128 = jnp.zeros((npad, 128), F32)
    ones_blk = jnp.ones((K, 128), F32)
    b1r = b1.reshape(1, -1)
    b2r = b2.reshape(1, -1)

    histp = _sc_hist(dst2d, ones_blk, zeros128).reshape(NC, npad, 128)
    v1, dinv = _tc1(histp, x, rb)
    y1p = _sc_scatter(v1, src2d, dst2d, zeros128).reshape(NC, npad, 128)
    v2 = _tc2(y1p, v1, dinv, W1, b1r, W2, rb)
    y2p = _sc_scatter(v2, src2d, dst2d, zeros128).reshape(NC, npad, 128)
    out = _tc3(y2p, v2, dinv, b2r, rb)
    return out
